# baseline (speedup 1.0000x reference)
import functools

import jax
import jax.numpy as jnp
from jax import lax
from jax.experimental import pallas as pl
from jax.experimental.pallas import tpu as pltpu
from jax.experimental.pallas import tpu_sc as plsc

_N = 10000
_E = 640000
_D = 128
_H = 64
_G = 64

_NW = 32
_CH = 128
_SLAB = 32
_NSL = 5
_NCH = _SLAB * _NSL
_EPW = _CH * _NCH
_EPAD = _NW * _EPW
_RPT = 640
_NP = 16 * _RPT
_DUMMY = _N

_NB = 10
_R = _N // _NB
_EPS = 1e-5
_PREC = lax.Precision.HIGHEST


def _seg_sum_sc(dh):
    che = 8192 // dh
    nsl = _EPW // (_SLAB * che)
    mesh = plsc.VectorSubcoreMesh(core_axis_name="c", subcore_axis_name="s")

    @functools.partial(
        pl.kernel,
        out_type=jax.ShapeDtypeStruct((2, _NP, dh), jnp.float32),
        mesh=mesh,
        compiler_params=pltpu.CompilerParams(use_tc_tiling_on_sc=False),
        scratch_types=[
            pltpu.VMEM((_SLAB, che), jnp.int32),
            pltpu.VMEM((_SLAB, che), jnp.int32),
            pltpu.VMEM((4, che, dh), jnp.float32),
            pltpu.VMEM_SHARED((_NP, dh), jnp.float32),
            pltpu.VMEM_SHARED((_N, dh), jnp.float32),
            pltpu.SemaphoreType.DMA,
            pltpu.SemaphoreType.DMA,
            pltpu.SemaphoreType.DMA,
            pltpu.SemaphoreType.DMA,
            pltpu.SemaphoreType.DMA,
            pltpu.SemaphoreType.DMA,
            pltpu.SemaphoreType.DMA,
            pltpu.SemaphoreType.DMA,
        ],
    )
    def k(h_hbm, src_hbm, dst_hbm, z_hbm, out_hbm, src_v, dst_v, rows_v,
          agg_sh, tab_sh, g0, g1, g2, g3, s0, s1, s2, s3):
        sg = [g0, g1, g2, g3]
        ss = [s0, s1, s2, s3]
        c = lax.axis_index("c")
        s = lax.axis_index("s")
        wid = s * 2 + c
        pltpu.sync_copy(z_hbm, agg_sh.at[pl.ds(s * _RPT, _RPT)])
        plsc.subcore_barrier()

        def gather(j, kk):
            return pltpu.make_async_copy(
                h_hbm.at[src_v.at[j]], rows_v.at[kk], sg[kk])

        def scatter(j, kk):
            return pltpu.make_async_copy(
                rows_v.at[kk], agg_sh.at[dst_v.at[j]], ss[kk])

        def slab(t, carry):
            pltpu.sync_copy(src_hbm.at[wid, pl.ds(t * _SLAB, _SLAB)], src_v)
            pltpu.sync_copy(dst_hbm.at[wid, pl.ds(t * _SLAB, _SLAB)], dst_v)
            for kk in range(4):
                gather(kk, kk).start()

            def grp(q, carry2):
                for kk in range(4):
                    gather(4 * q + kk, kk).wait()
                    scatter(4 * q + kk, kk).start(add=True)
                for kk in range(4):
                    scatter(4 * q + kk, kk).wait()
                    gather(4 * (q + 1) + kk, kk).start()
                return carry2

            lax.fori_loop(0, _SLAB // 4 - 1, grp, carry)
            last = _SLAB - 4
            for kk in range(4):
                gather(last + kk, kk).wait()
                scatter(last + kk, kk).start(add=True)
            for kk in range(4):
                scatter(last + kk, kk).wait()
            return carry

        lax.fori_loop(0, nsl, slab, 0)
        plsc.subcore_barrier()
        pltpu.sync_copy(agg_sh.at[pl.ds(s * _RPT, _RPT)],
                        out_hbm.at[c, pl.ds(s * _RPT, _RPT)])

    return k, che


def _bn_cols(hp, gamma, beta):
    mean = jnp.sum(hp, axis=0, keepdims=True) * (1.0 / _N)
    var = jnp.sum(hp * hp, axis=0, keepdims=True) * (1.0 / _N) - mean * mean
    scale = gamma * lax.rsqrt(var + _EPS)
    return hp * scale + (beta - mean * scale)


def _pre1(x, W):
    def body(x_ref, W_ref, y_ref):
        y_ref[...] = jnp.dot(x_ref[...], W_ref[...], precision=_PREC,
                             preferred_element_type=jnp.float32)

    return pl.pallas_call(
        body,
        out_shape=jax.ShapeDtypeStruct((_N, _H), jnp.float32),
    )(x, W)


def _dense(y, a0, a1, p, W1n, res):
    has_res = res is not None

    def body(*refs):
        (y_ref, a0_ref, a1_ref, b1_ref, g1_ref, be1_ref, W2_ref, b2_ref,
         g2_ref, be2_ref, W1n_ref) = refs[:11]
        rest = refs[11:]
        if has_res:
            r_ref = rest[0]
            rest = rest[1:]
        h_ref, yn_ref = rest
        hp = y_ref[...] + a0_ref[...] + a1_ref[...] + b1_ref[...]
        r1 = jnp.maximum(_bn_cols(hp, g1_ref[...], be1_ref[...]), 0.0)
        hp2 = jnp.dot(r1, W2_ref[...], precision=_PREC,
                      preferred_element_type=jnp.float32) + b2_ref[...]
        h = jnp.maximum(_bn_cols(hp2, g2_ref[...], be2_ref[...]), 0.0)
        if has_res:
            h = h + r_ref[...]
        h_ref[...] = h
        yn_ref[...] = jnp.dot(h, W1n_ref[...], precision=_PREC,
                              preferred_element_type=jnp.float32)

    args = [y, a0, a1, p['b1'].reshape(1, -1), p['g1'].reshape(1, -1),
            p['be1'].reshape(1, -1), p['W2'], p['b2'].reshape(1, -1),
            p['g2'].reshape(1, -1), p['be2'].reshape(1, -1), W1n]
    if has_res:
        args.append(res)

    return pl.pallas_call(
        body,
        out_shape=[
            jax.ShapeDtypeStruct((_N, _H), jnp.float32),
            jax.ShapeDtypeStruct((_N, _H), jnp.float32),
        ],
    )(*args)


def _final(y, a0, a1, p, res, batch_f, hp):
    names = ['head_color', 'head_size', 'head_ground', 'head_struct']
    douts = [16, 16, 8, 32]

    def body(*refs):
        (y_ref, a0_ref, a1_ref, b1_ref, g1_ref, be1_ref, W2_ref, b2_ref,
         g2_ref, be2_ref, r_ref, bat_ref) = refs[:12]
        hrefs = refs[12:12 + 4 * len(names)]
        outs = refs[12 + 4 * len(names):]
        hp_ = y_ref[...] + a0_ref[...] + a1_ref[...] + b1_ref[...]
        r1 = jnp.maximum(_bn_cols(hp_, g1_ref[...], be1_ref[...]), 0.0)
        hp2 = jnp.dot(r1, W2_ref[...], precision=_PREC,
                      preferred_element_type=jnp.float32) + b2_ref[...]
        h = jnp.maximum(_bn_cols(hp2, g2_ref[...], be2_ref[...]), 0.0)
        h = h + r_ref[...]
        seg = lax.broadcasted_iota(jnp.int32, (_N, _G), 1).astype(jnp.float32)
        onehot = jnp.where(bat_ref[...] == seg, 1.0, 0.0)
        g = lax.dot_general(onehot, h, (((0,), (0,)), ((), ())),
                            precision=_PREC,
                            preferred_element_type=jnp.float32)
        for n in range(len(names)):
            W1, b1, W2, b2 = hrefs[4 * n:4 * (n + 1)]
            rr = jnp.maximum(
                jnp.dot(g, W1[...], precision=_PREC,
                        preferred_element_type=jnp.float32) + b1[...], 0.0)
            z = jnp.dot(rr, W2[...], precision=_PREC,
                        preferred_element_type=jnp.float32) + b2[...]
            nrm = jnp.sqrt(jnp.sum(z * z, axis=1, keepdims=True))
            outs[n][...] = z / jnp.maximum(nrm, 1e-12)

    args = [y, a0, a1, p['b1'].reshape(1, -1), p['g1'].reshape(1, -1),
            p['be1'].reshape(1, -1), p['W2'], p['b2'].reshape(1, -1),
            p['g2'].reshape(1, -1), p['be2'].reshape(1, -1), res, batch_f]
    for n in names:
        q = hp[n]
        args += [q['W1'], q['b1'].reshape(1, -1), q['W2'],
                 q['b2'].reshape(1, -1)]

    return pl.pallas_call(
        body,
        out_shape=[jax.ShapeDtypeStruct((_G, d), jnp.float32) for d in douts],
    )(*args)


def kernel(x, edge_index, batch, params):
    src = edge_index[0]
    dst = edge_index[1]
    npad = _EPW - _E // _NW
    dummy_dst = jnp.broadcast_to(
        _DUMMY + (jnp.arange(npad, dtype=jnp.int32) % (_NP - _N)),
        (_NW, npad))
    pad_src = jnp.concatenate(
        [src.reshape(_NW, _E // _NW),
         jnp.zeros((_NW, npad), jnp.int32)], axis=1)
    pad_dst = jnp.concatenate(
        [dst.reshape(_NW, _E // _NW), dummy_dst], axis=1)
    zblk = jnp.zeros((_RPT, _H), jnp.float32)
    batch_f = batch.astype(jnp.float32).reshape(_N, 1)

    seg, che = _seg_sum_sc(_H)
    srcp = pad_src.reshape(_NW, _EPW // che, che)
    dstp = pad_dst.reshape(_NW, _EPW // che, che)

    def agg2(y):
        agg = seg(y, srcp, dstp, zblk)
        a0 = lax.slice(agg, (0, 0, 0), (1, _N, _H)).reshape(_N, _H)
        a1 = lax.slice(agg, (1, 0, 0), (2, _N, _H)).reshape(_N, _H)
        return a0, a1

    y1 = _pre1(x, params['conv1']['W1'])
    a0, a1 = agg2(y1)
    h1, y2 = _dense(y1, a0, a1, params['conv1'], params['conv2']['W1'], None)
    a0, a1 = agg2(y2)
    h2, y3 = _dense(y2, a0, a1, params['conv2'], params['conv3']['W1'], h1)
    a0, a1 = agg2(y3)
    return _final(y3, a0, a1, params['conv3'], h2, batch_f, params)


# speedup vs baseline: 1.8581x; 1.8581x over previous
import functools

import jax
import jax.numpy as jnp
from jax import lax
from jax.experimental import pallas as pl
from jax.experimental.pallas import tpu as pltpu
from jax.experimental.pallas import tpu_sc as plsc

_N = 10000
_E = 640000
_D = 128
_H = 64
_G = 64

_NW = 32
_CH = 128
_SLAB = 32
_NSL = 5
_NCH = _SLAB * _NSL
_EPW = _CH * _NCH
_EPAD = _NW * _EPW
_RPT = 640
_NP = 16 * _RPT
_DUMMY = _N

_NB = 10
_R = _N // _NB
_EPS = 1e-5
_PREC = lax.Precision.HIGHEST


def _seg_sum_sc(dh):
    che = 8192 // dh
    nsl = _EPW // (_SLAB * che)
    mesh = plsc.VectorSubcoreMesh(core_axis_name="c", subcore_axis_name="s")

    @functools.partial(
        pl.kernel,
        out_type=jax.ShapeDtypeStruct((2, _NP, dh), jnp.float32),
        mesh=mesh,
        compiler_params=pltpu.CompilerParams(use_tc_tiling_on_sc=False),
        scratch_types=[
            pltpu.VMEM((_SLAB, che), jnp.int32),
            pltpu.VMEM((_SLAB, che), jnp.int32),
            pltpu.VMEM((4, che, dh), jnp.float32),
            pltpu.VMEM_SHARED((_NP, dh), jnp.float32),
            pltpu.VMEM_SHARED((_N, dh), jnp.float32),
            pltpu.SemaphoreType.DMA,
            pltpu.SemaphoreType.DMA,
            pltpu.SemaphoreType.DMA,
            pltpu.SemaphoreType.DMA,
            pltpu.SemaphoreType.DMA,
            pltpu.SemaphoreType.DMA,
            pltpu.SemaphoreType.DMA,
            pltpu.SemaphoreType.DMA,
        ],
    )
    def k(h_hbm, src_hbm, dst_hbm, z_hbm, out_hbm, src_v, dst_v, rows_v,
          agg_sh, tab_sh, g0, g1, g2, g3, s0, s1, s2, s3):
        sg = [g0, g1, g2, g3]
        ss = [s0, s1, s2, s3]
        c = lax.axis_index("c")
        s = lax.axis_index("s")
        wid = s * 2 + c
        pltpu.sync_copy(z_hbm, agg_sh.at[pl.ds(s * _RPT, _RPT)])
        pltpu.sync_copy(h_hbm.at[pl.ds(s * (_N // 16), _N // 16)],
                        tab_sh.at[pl.ds(s * (_N // 16), _N // 16)])
        plsc.subcore_barrier()

        def gather(j, kk):
            return pltpu.make_async_copy(
                tab_sh.at[src_v.at[j]], rows_v.at[kk], sg[kk])

        def scatter(j, kk):
            return pltpu.make_async_copy(
                rows_v.at[kk], agg_sh.at[dst_v.at[j]], ss[kk])

        def slab(t, carry):
            pltpu.sync_copy(src_hbm.at[wid, pl.ds(t * _SLAB, _SLAB)], src_v)
            pltpu.sync_copy(dst_hbm.at[wid, pl.ds(t * _SLAB, _SLAB)], dst_v)
            for kk in range(4):
                gather(kk, kk).start()

            def grp(q, carry2):
                for kk in range(4):
                    gather(4 * q + kk, kk).wait()
                    scatter(4 * q + kk, kk).start(add=True)
                for kk in range(4):
                    scatter(4 * q + kk, kk).wait()
                    gather(4 * (q + 1) + kk, kk).start()
                return carry2

            lax.fori_loop(0, _SLAB // 4 - 1, grp, carry)
            last = _SLAB - 4
            for kk in range(4):
                gather(last + kk, kk).wait()
                scatter(last + kk, kk).start(add=True)
            for kk in range(4):
                scatter(last + kk, kk).wait()
            return carry

        lax.fori_loop(0, nsl, slab, 0)
        plsc.subcore_barrier()
        pltpu.sync_copy(agg_sh.at[pl.ds(s * _RPT, _RPT)],
                        out_hbm.at[c, pl.ds(s * _RPT, _RPT)])

    return k, che


def _bn_cols(hp, gamma, beta):
    mean = jnp.sum(hp, axis=0, keepdims=True) * (1.0 / _N)
    var = jnp.sum(hp * hp, axis=0, keepdims=True) * (1.0 / _N) - mean * mean
    scale = gamma * lax.rsqrt(var + _EPS)
    return hp * scale + (beta - mean * scale)


def _bn_scale_shift(st_ref, gamma, beta):
    mean = st_ref[0:1, :] * (1.0 / _N)
    var = st_ref[1:2, :] * (1.0 / _N) - mean * mean
    scale = gamma * lax.rsqrt(var + _EPS)
    shift = beta - mean * scale
    return scale, shift


def _pre1(x, W):
    def body(x_ref, W_ref, y_ref):
        y_ref[...] = jnp.dot(x_ref[...], W_ref[...], precision=_PREC,
                             preferred_element_type=jnp.float32)

    return pl.pallas_call(
        body,
        grid=(_NB,),
        in_specs=[
            pl.BlockSpec((_R, _D), lambda i: (i, 0)),
            pl.BlockSpec((_D, _H), lambda i: (0, 0)),
        ],
        out_specs=pl.BlockSpec((_R, _H), lambda i: (i, 0)),
        out_shape=jax.ShapeDtypeStruct((_N, _H), jnp.float32),
    )(x, W)


def _dense(y, a0, a1, p, W1n, res):
    has_res = res is not None

    def body(*refs):
        (y_ref, a0_ref, a1_ref, b1_ref, g1_ref, be1_ref, W2_ref, b2_ref,
         g2_ref, be2_ref, W1n_ref) = refs[:11]
        rest = refs[11:]
        if has_res:
            r_ref = rest[0]
            rest = rest[1:]
        h_ref, yn_ref, hp_scr, hp2_scr, st1, st2 = rest
        ph = pl.program_id(0)
        i = pl.program_id(1)

        @pl.when(jnp.logical_and(ph == 0, i == 0))
        def _():
            st1[...] = jnp.zeros_like(st1)
            st2[...] = jnp.zeros_like(st2)

        @pl.when(ph == 0)
        def _():
            hp = y_ref[...] + a0_ref[...] + a1_ref[...] + b1_ref[...]
            hp_scr[pl.ds(i * _R, _R), :] = hp
            st1[0:1, :] += jnp.sum(hp, axis=0, keepdims=True)
            st1[1:2, :] += jnp.sum(hp * hp, axis=0, keepdims=True)

        @pl.when(ph == 1)
        def _():
            scale, shift = _bn_scale_shift(st1, g1_ref[...], be1_ref[...])
            r1 = jnp.maximum(hp_scr[pl.ds(i * _R, _R), :] * scale + shift,
                             0.0)
            hp2 = jnp.dot(r1, W2_ref[...], precision=_PREC,
                          preferred_element_type=jnp.float32) + b2_ref[...]
            hp2_scr[pl.ds(i * _R, _R), :] = hp2
            st2[0:1, :] += jnp.sum(hp2, axis=0, keepdims=True)
            st2[1:2, :] += jnp.sum(hp2 * hp2, axis=0, keepdims=True)

        @pl.when(ph == 2)
        def _():
            scale, shift = _bn_scale_shift(st2, g2_ref[...], be2_ref[...])
            h = jnp.maximum(hp2_scr[pl.ds(i * _R, _R), :] * scale + shift,
                            0.0)
            if has_res:
                h = h + r_ref[...]
            h_ref[...] = h
            yn_ref[...] = jnp.dot(h, W1n_ref[...], precision=_PREC,
                                  preferred_element_type=jnp.float32)

    full = lambda p, i: (0, 0)
    p0 = lambda p, i: (jnp.where(p == 0, i, 0), 0)
    p2 = lambda p, i: (jnp.where(p == 2, i, 0), 0)
    out_m = lambda p, i: (i, 0)
    in_specs = [
        pl.BlockSpec((_R, _H), p0),
        pl.BlockSpec((_R, _H), p0),
        pl.BlockSpec((_R, _H), p0),
        pl.BlockSpec((1, _H), full),
        pl.BlockSpec((1, _H), full),
        pl.BlockSpec((1, _H), full),
        pl.BlockSpec((_H, _H), full),
        pl.BlockSpec((1, _H), full),
        pl.BlockSpec((1, _H), full),
        pl.BlockSpec((1, _H), full),
        pl.BlockSpec((_H, _H), full),
    ]
    args = [y, a0, a1, p['b1'].reshape(1, -1), p['g1'].reshape(1, -1),
            p['be1'].reshape(1, -1), p['W2'], p['b2'].reshape(1, -1),
            p['g2'].reshape(1, -1), p['be2'].reshape(1, -1), W1n]
    if has_res:
        in_specs.append(pl.BlockSpec((_R, _H), p2))
        args.append(res)

    return pl.pallas_call(
        body,
        grid=(3, _NB),
        in_specs=in_specs,
        out_specs=[
            pl.BlockSpec((_R, _H), out_m),
            pl.BlockSpec((_R, _H), out_m),
        ],
        out_shape=[
            jax.ShapeDtypeStruct((_N, _H), jnp.float32),
            jax.ShapeDtypeStruct((_N, _H), jnp.float32),
        ],
        scratch_shapes=[
            pltpu.VMEM((_N, _H), jnp.float32),
            pltpu.VMEM((_N, _H), jnp.float32),
            pltpu.VMEM((8, _H), jnp.float32),
            pltpu.VMEM((8, _H), jnp.float32),
        ],
    )(*args)


def _final(y, a0, a1, p, res, batch_f, hp):
    names = ['head_color', 'head_size', 'head_ground', 'head_struct']
    douts = [16, 16, 8, 32]

    def body(*refs):
        (y_ref, a0_ref, a1_ref, b1_ref, g1_ref, be1_ref, W2_ref, b2_ref,
         g2_ref, be2_ref, r_ref, bat_ref) = refs[:12]
        hrefs = refs[12:12 + 4 * len(names)]
        outs = refs[12 + 4 * len(names):]
        hp_ = y_ref[...] + a0_ref[...] + a1_ref[...] + b1_ref[...]
        r1 = jnp.maximum(_bn_cols(hp_, g1_ref[...], be1_ref[...]), 0.0)
        hp2 = jnp.dot(r1, W2_ref[...], precision=_PREC,
                      preferred_element_type=jnp.float32) + b2_ref[...]
        h = jnp.maximum(_bn_cols(hp2, g2_ref[...], be2_ref[...]), 0.0)
        h = h + r_ref[...]
        seg = lax.broadcasted_iota(jnp.int32, (_N, _G), 1).astype(jnp.float32)
        onehot = jnp.where(bat_ref[...] == seg, 1.0, 0.0)
        g = lax.dot_general(onehot, h, (((0,), (0,)), ((), ())),
                            precision=_PREC,
                            preferred_element_type=jnp.float32)
        for n in range(len(names)):
            W1, b1, W2, b2 = hrefs[4 * n:4 * (n + 1)]
            rr = jnp.maximum(
                jnp.dot(g, W1[...], precision=_PREC,
                        preferred_element_type=jnp.float32) + b1[...], 0.0)
            z = jnp.dot(rr, W2[...], precision=_PREC,
                        preferred_element_type=jnp.float32) + b2[...]
            nrm = jnp.sqrt(jnp.sum(z * z, axis=1, keepdims=True))
            outs[n][...] = z / jnp.maximum(nrm, 1e-12)

    args = [y, a0, a1, p['b1'].reshape(1, -1), p['g1'].reshape(1, -1),
            p['be1'].reshape(1, -1), p['W2'], p['b2'].reshape(1, -1),
            p['g2'].reshape(1, -1), p['be2'].reshape(1, -1), res, batch_f]
    for n in names:
        q = hp[n]
        args += [q['W1'], q['b1'].reshape(1, -1), q['W2'],
                 q['b2'].reshape(1, -1)]

    return pl.pallas_call(
        body,
        out_shape=[jax.ShapeDtypeStruct((_G, d), jnp.float32) for d in douts],
    )(*args)


def kernel(x, edge_index, batch, params):
    src = edge_index[0]
    dst = edge_index[1]
    npad = _EPW - _E // _NW
    dummy_dst = jnp.broadcast_to(
        _DUMMY + (jnp.arange(npad, dtype=jnp.int32) % (_NP - _N)),
        (_NW, npad))
    pad_src = jnp.concatenate(
        [src.reshape(_NW, _E // _NW),
         jnp.zeros((_NW, npad), jnp.int32)], axis=1)
    pad_dst = jnp.concatenate(
        [dst.reshape(_NW, _E // _NW), dummy_dst], axis=1)
    zblk = jnp.zeros((_RPT, _H), jnp.float32)
    batch_f = batch.astype(jnp.float32).reshape(_N, 1)

    seg, che = _seg_sum_sc(_H)
    srcp = pad_src.reshape(_NW, _EPW // che, che)
    dstp = pad_dst.reshape(_NW, _EPW // che, che)

    def agg2(y):
        agg = seg(y, srcp, dstp, zblk)
        a0 = lax.slice(agg, (0, 0, 0), (1, _N, _H)).reshape(_N, _H)
        a1 = lax.slice(agg, (1, 0, 0), (2, _N, _H)).reshape(_N, _H)
        return a0, a1

    y1 = _pre1(x, params['conv1']['W1'])
    a0, a1 = agg2(y1)
    h1, y2 = _dense(y1, a0, a1, params['conv1'], params['conv2']['W1'], None)
    a0, a1 = agg2(y2)
    h2, y3 = _dense(y2, a0, a1, params['conv2'], params['conv3']['W1'], h1)
    a0, a1 = agg2(y3)
    return _final(y3, a0, a1, params['conv3'], h2, batch_f, params)


# speedup vs baseline: 1.9131x; 1.0296x over previous
import functools

import jax
import jax.numpy as jnp
from jax import lax
from jax.experimental import pallas as pl
from jax.experimental.pallas import tpu as pltpu
from jax.experimental.pallas import tpu_sc as plsc

_N = 10000
_E = 640000
_D = 128
_H = 64
_G = 64

_NW = 32
_CH = 128
_SLAB = 32
_NSL = 5
_NCH = _SLAB * _NSL
_EPW = _CH * _NCH
_EPAD = _NW * _EPW
_RPT = 640
_NP = 16 * _RPT
_DUMMY = _N

_NB = 10
_R = _N // _NB
_EPS = 1e-5
_PREC = lax.Precision.HIGHEST


def _seg_sum_sc(dh):
    che = 8192 // dh
    nsl = _EPW // (_SLAB * che)
    mesh = plsc.VectorSubcoreMesh(core_axis_name="c", subcore_axis_name="s")

    @functools.partial(
        pl.kernel,
        out_type=jax.ShapeDtypeStruct((2, _NP, dh), jnp.float32),
        mesh=mesh,
        compiler_params=pltpu.CompilerParams(use_tc_tiling_on_sc=False),
        scratch_types=[
            pltpu.VMEM((_SLAB, che), jnp.int32),
            pltpu.VMEM((_SLAB, che), jnp.int32),
            pltpu.VMEM((4, che, dh), jnp.float32),
            pltpu.VMEM_SHARED((_NP, dh), jnp.float32),
            pltpu.VMEM_SHARED((_N, dh), jnp.float32),
            pltpu.SemaphoreType.DMA,
            pltpu.SemaphoreType.DMA,
            pltpu.SemaphoreType.DMA,
            pltpu.SemaphoreType.DMA,
            pltpu.SemaphoreType.DMA,
            pltpu.SemaphoreType.DMA,
            pltpu.SemaphoreType.DMA,
            pltpu.SemaphoreType.DMA,
        ],
    )
    def k(h_hbm, src_hbm, dst_hbm, z_hbm, out_hbm, src_v, dst_v, rows_v,
          agg_sh, tab_sh, g0, g1, g2, g3, s0, s1, s2, s3):
        sg = [g0, g1, g2, g3]
        ss = [s0, s1, s2, s3]
        c = lax.axis_index("c")
        s = lax.axis_index("s")
        wid = s * 2 + c
        zc = pltpu.make_async_copy(
            z_hbm, agg_sh.at[pl.ds(s * _RPT, _RPT)], sg[0])
        tc = pltpu.make_async_copy(
            h_hbm.at[pl.ds(s * (_N // 16), _N // 16)],
            tab_sh.at[pl.ds(s * (_N // 16), _N // 16)], sg[1])
        zc.start()
        tc.start()
        zc.wait()
        tc.wait()
        plsc.subcore_barrier()

        def gather(j, kk):
            return pltpu.make_async_copy(
                tab_sh.at[src_v.at[j]], rows_v.at[kk], sg[kk])

        def scatter(j, kk):
            return pltpu.make_async_copy(
                rows_v.at[kk], agg_sh.at[dst_v.at[j]], ss[kk])

        def slab(t, carry):
            pltpu.sync_copy(src_hbm.at[wid, pl.ds(t * _SLAB, _SLAB)], src_v)
            pltpu.sync_copy(dst_hbm.at[wid, pl.ds(t * _SLAB, _SLAB)], dst_v)
            for kk in range(4):
                gather(kk, kk).start()

            def grp(q, carry2):
                for kk in range(4):
                    gather(4 * q + kk, kk).wait()
                    scatter(4 * q + kk, kk).start(add=True)
                for kk in range(4):
                    scatter(4 * q + kk, kk).wait()
                    gather(4 * (q + 1) + kk, kk).start()
                return carry2

            lax.fori_loop(0, _SLAB // 4 - 1, grp, carry)
            last = _SLAB - 4
            for kk in range(4):
                gather(last + kk, kk).wait()
                scatter(last + kk, kk).start(add=True)
            for kk in range(4):
                scatter(last + kk, kk).wait()
            return carry

        lax.fori_loop(0, nsl, slab, 0)
        plsc.subcore_barrier()
        pltpu.sync_copy(agg_sh.at[pl.ds(s * _RPT, _RPT)],
                        out_hbm.at[c, pl.ds(s * _RPT, _RPT)])

    return k, che


def _bn_cols(hp, gamma, beta):
    mean = jnp.sum(hp, axis=0, keepdims=True) * (1.0 / _N)
    var = jnp.sum(hp * hp, axis=0, keepdims=True) * (1.0 / _N) - mean * mean
    scale = gamma * lax.rsqrt(var + _EPS)
    return hp * scale + (beta - mean * scale)


def _pre1(x, W):
    def body(x_ref, W_ref, y_ref):
        y_ref[...] = jnp.dot(x_ref[...], W_ref[...], precision=_PREC,
                             preferred_element_type=jnp.float32)

    return pl.pallas_call(
        body,
        out_shape=jax.ShapeDtypeStruct((_N, _H), jnp.float32),
    )(x, W)


def _dense(y, a0, a1, p, W1n, res):
    has_res = res is not None

    def body(*refs):
        (y_ref, a0_ref, a1_ref, b1_ref, g1_ref, be1_ref, W2_ref, b2_ref,
         g2_ref, be2_ref, W1n_ref) = refs[:11]
        rest = refs[11:]
        if has_res:
            r_ref = rest[0]
            rest = rest[1:]
        h_ref, yn_ref = rest
        hp = y_ref[...] + a0_ref[...] + a1_ref[...] + b1_ref[...]
        r1 = jnp.maximum(_bn_cols(hp, g1_ref[...], be1_ref[...]), 0.0)
        hp2 = jnp.dot(r1, W2_ref[...], precision=_PREC,
                      preferred_element_type=jnp.float32) + b2_ref[...]
        h = jnp.maximum(_bn_cols(hp2, g2_ref[...], be2_ref[...]), 0.0)
        if has_res:
            h = h + r_ref[...]
        h_ref[...] = h
        yn_ref[...] = jnp.dot(h, W1n_ref[...], precision=_PREC,
                              preferred_element_type=jnp.float32)

    args = [y, a0, a1, p['b1'].reshape(1, -1), p['g1'].reshape(1, -1),
            p['be1'].reshape(1, -1), p['W2'], p['b2'].reshape(1, -1),
            p['g2'].reshape(1, -1), p['be2'].reshape(1, -1), W1n]
    if has_res:
        args.append(res)

    return pl.pallas_call(
        body,
        out_shape=[
            jax.ShapeDtypeStruct((_N, _H), jnp.float32),
            jax.ShapeDtypeStruct((_N, _H), jnp.float32),
        ],
    )(*args)


def _final(y, a0, a1, p, res, batch_f, hp):
    names = ['head_color', 'head_size', 'head_ground', 'head_struct']
    douts = [16, 16, 8, 32]

    def body(*refs):
        (y_ref, a0_ref, a1_ref, b1_ref, g1_ref, be1_ref, W2_ref, b2_ref,
         g2_ref, be2_ref, r_ref, bat_ref) = refs[:12]
        hrefs = refs[12:12 + 4 * len(names)]
        outs = refs[12 + 4 * len(names):]
        hp_ = y_ref[...] + a0_ref[...] + a1_ref[...] + b1_ref[...]
        r1 = jnp.maximum(_bn_cols(hp_, g1_ref[...], be1_ref[...]), 0.0)
        hp2 = jnp.dot(r1, W2_ref[...], precision=_PREC,
                      preferred_element_type=jnp.float32) + b2_ref[...]
        h = jnp.maximum(_bn_cols(hp2, g2_ref[...], be2_ref[...]), 0.0)
        h = h + r_ref[...]
        seg = lax.broadcasted_iota(jnp.int32, (_N, _G), 1).astype(jnp.float32)
        onehot = jnp.where(bat_ref[...] == seg, 1.0, 0.0)
        g = lax.dot_general(onehot, h, (((0,), (0,)), ((), ())),
                            precision=_PREC,
                            preferred_element_type=jnp.float32)
        for n in range(len(names)):
            W1, b1, W2, b2 = hrefs[4 * n:4 * (n + 1)]
            rr = jnp.maximum(
                jnp.dot(g, W1[...], precision=_PREC,
                        preferred_element_type=jnp.float32) + b1[...], 0.0)
            z = jnp.dot(rr, W2[...], precision=_PREC,
                        preferred_element_type=jnp.float32) + b2[...]
            nrm = jnp.sqrt(jnp.sum(z * z, axis=1, keepdims=True))
            outs[n][...] = z / jnp.maximum(nrm, 1e-12)

    args = [y, a0, a1, p['b1'].reshape(1, -1), p['g1'].reshape(1, -1),
            p['be1'].reshape(1, -1), p['W2'], p['b2'].reshape(1, -1),
            p['g2'].reshape(1, -1), p['be2'].reshape(1, -1), res, batch_f]
    for n in names:
        q = hp[n]
        args += [q['W1'], q['b1'].reshape(1, -1), q['W2'],
                 q['b2'].reshape(1, -1)]

    return pl.pallas_call(
        body,
        out_shape=[jax.ShapeDtypeStruct((_G, d), jnp.float32) for d in douts],
    )(*args)


def kernel(x, edge_index, batch, params):
    src = edge_index[0]
    dst = edge_index[1]
    npad = _EPW - _E // _NW
    dummy_dst = jnp.broadcast_to(
        _DUMMY + (jnp.arange(npad, dtype=jnp.int32) % (_NP - _N)),
        (_NW, npad))
    pad_src = jnp.concatenate(
        [src.reshape(_NW, _E // _NW),
         jnp.zeros((_NW, npad), jnp.int32)], axis=1)
    pad_dst = jnp.concatenate(
        [dst.reshape(_NW, _E // _NW), dummy_dst], axis=1)
    zblk = jnp.zeros((_RPT, _H), jnp.float32)
    batch_f = batch.astype(jnp.float32).reshape(_N, 1)

    seg, che = _seg_sum_sc(_H)
    srcp = pad_src.reshape(_NW, _EPW // che, che)
    dstp = pad_dst.reshape(_NW, _EPW // che, che)

    def agg2(y):
        agg = seg(y, srcp, dstp, zblk)
        a0 = lax.slice(agg, (0, 0, 0), (1, _N, _H)).reshape(_N, _H)
        a1 = lax.slice(agg, (1, 0, 0), (2, _N, _H)).reshape(_N, _H)
        return a0, a1

    y1 = _pre1(x, params['conv1']['W1'])
    a0, a1 = agg2(y1)
    h1, y2 = _dense(y1, a0, a1, params['conv1'], params['conv2']['W1'], None)
    a0, a1 = agg2(y2)
    h2, y3 = _dense(y2, a0, a1, params['conv2'], params['conv3']['W1'], h1)
    a0, a1 = agg2(y3)
    return _final(y3, a0, a1, params['conv3'], h2, batch_f, params)
